# baseline (speedup 1.0000x reference)
import functools

import jax
import jax.numpy as jnp
from jax.experimental import pallas as pl
from jax.experimental.pallas import tpu as pltpu


def _round_up(v, m):
    return ((v + m - 1) // m) * m


def _convN_kernel(v_ref, w_ref, b_ref, o_ref, *, kh, kw, stride, cin, cout,
                  ho_n, wo_n):
    V = v_ref[0]
    S = stride * cin
    win = (kw - 1) * cin + cin
    ndj = (kw * cin + S - 1) // S
    pieces = []
    left = win
    for dj in range(ndj):
        wdt = min(S, left)
        pieces.append(V[:, dj:dj + wo_n, :wdt])
        left -= wdt
    T = jnp.concatenate(pieces, axis=-1)
    w = w_ref[...]
    b = b_ref[...]
    for ho in range(ho_n):
        lhs = jnp.concatenate([T[stride * ho + i] for i in range(kh)],
                              axis=-1)
        acc = jnp.dot(lhs, w, preferred_element_type=jnp.float32)
        out = jnp.maximum(acc[:, :cout] + b, 0.0)
        o_ref[0, ho] = out.astype(o_ref.dtype)


def _convN_fused(x, wmat, b2, *, cout, ksize, stride, cin=None,
                 prearranged=False):
    n, h = x.shape[:2]
    if prearranged:
        c = cin
        w = x.shape[2] * x.shape[3] // c
        v = x
    else:
        w, c = x.shape[2], x.shape[3]
        if w % stride:
            x = jnp.pad(x, ((0, 0), (0, 0), (0, stride - w % stride),
                            (0, 0)))
        v = x.reshape(n, h, (w + stride - 1) // stride * stride // stride,
                      stride * c)
    ho = (h - ksize) // stride + 1
    wo = (w - ksize) // stride + 1
    kern = functools.partial(_convN_kernel, kh=ksize, kw=ksize, stride=stride,
                             cin=c, cout=cout, ho_n=ho, wo_n=wo)
    return pl.pallas_call(
        kern,
        out_shape=jax.ShapeDtypeStruct((n, ho, wo, cout), jnp.bfloat16),
        grid=(n,),
        in_specs=[
            pl.BlockSpec((1,) + v.shape[1:], lambda i: (i, 0, 0, 0)),
            pl.BlockSpec(wmat.shape, lambda i: (0, 0)),
            pl.BlockSpec((1, cout), lambda i: (0, 0)),
        ],
        out_specs=pl.BlockSpec((1, ho, wo, cout), lambda i: (i, 0, 0, 0)),
        compiler_params=pltpu.CompilerParams(
            dimension_semantics=("parallel",)),
    )(v, wmat, b2[:, :cout])


def _conv1_kernel(xm_ref, g_ref, w_ref, b_ref, o_ref, *, ho_n, wo_n, cin,
                  cout, split_out):
    c_, h_, w_ = xm_ref.shape[1:]
    xm = xm_ref[0].reshape(c_ * h_, w_).astype(jnp.bfloat16)
    xt = jax.lax.dot_general(xm, g_ref[...], (((0,), (0,)), ((), ())),
                             preferred_element_type=jnp.float32)
    xp = xt.reshape(xt.shape[0] // 2, 2, xt.shape[1])
    w = w_ref[...]
    b = b_ref[...]
    kwin = 5 * cin
    for ho in range(ho_n):
        base = 2 * cin * ho
        pieces = []
        for j in range(5):
            p, dj = j % 2, j // 2
            pieces.append(xp[dj:dj + wo_n, p, base:base + kwin])
        lhs = jnp.concatenate(pieces, axis=-1)
        acc = jnp.dot(lhs.astype(jnp.bfloat16), w,
                      preferred_element_type=jnp.float32)
        out = jnp.maximum(acc[:, :cout] + b, 0.0).astype(o_ref.dtype)
        if split_out:
            r2 = out.reshape(wo_n // 2, 2, cout)
            o_ref[0, ho, :, :cout] = r2[:, 0, :]
            o_ref[0, ho, :, cout:] = r2[:, 1, :]
        else:
            o_ref[0, ho] = out


def _conv1_fused(x_nchw, wmat, b2, *, cout=24):
    n, c, h, w = x_nchw.shape
    ho = (h - 5) // 2 + 1
    wo = (w - 5) // 2 + 1
    xm = x_nchw
    if w % 2:
        xm = jnp.pad(xm, ((0, 0), (0, 0), (0, 0), (0, 1)))
        w += 1
    split_out = wo % 2 == 0
    cc = jnp.arange(c)
    hh = jnp.arange(h)
    g = ((hh[None, :, None, None] == hh[None, None, :, None])
         & (cc[:, None, None, None] == cc[None, None, None, :]))
    g = g.reshape(c * h, h * c).astype(jnp.bfloat16)
    wmat = (wmat.reshape(5, 5, c, wmat.shape[1])
            .transpose(1, 0, 2, 3).reshape(25 * c, wmat.shape[1]))
    kern = functools.partial(_conv1_kernel, ho_n=ho, wo_n=wo, cin=c,
                             cout=cout, split_out=split_out)
    if split_out:
        oshape = (n, ho, wo // 2, 2 * cout)
    else:
        oshape = (n, ho, wo, cout)
    out = pl.pallas_call(
        kern,
        out_shape=jax.ShapeDtypeStruct(oshape, jnp.bfloat16),
        grid=(n,),
        in_specs=[
            pl.BlockSpec((1, c, h, w), lambda i: (i, 0, 0, 0)),
            pl.BlockSpec(g.shape, lambda i: (0, 0)),
            pl.BlockSpec(wmat.shape, lambda i: (0, 0)),
            pl.BlockSpec((1, cout), lambda i: (0, 0)),
        ],
        out_specs=pl.BlockSpec((1,) + oshape[1:], lambda i: (i, 0, 0, 0)),
        compiler_params=pltpu.CompilerParams(
            dimension_semantics=("parallel",)),
    )(xm, g, wmat, b2[:, :cout])
    return out, split_out


def _fc_kernel(x_ref, w1_ref, b1_ref, w2_ref, b2_ref, w3_ref, b3_ref,
               o_ref, acc_ref):
    @pl.when(pl.program_id(1) == 0)
    def _():
        acc_ref[...] = jnp.zeros_like(acc_ref)

    acc_ref[...] += jnp.dot(x_ref[...], w1_ref[...],
                            preferred_element_type=jnp.float32)

    @pl.when(pl.program_id(1) == pl.num_programs(1) - 1)
    def _():
        h = jnp.maximum(acc_ref[...] + b1_ref[...], 0.0).astype(jnp.bfloat16)
        h = jnp.dot(h, w2_ref[...], preferred_element_type=jnp.float32)
        h = jnp.maximum(h + b2_ref[...], 0.0).astype(jnp.bfloat16)
        h = jnp.dot(h, w3_ref[...], preferred_element_type=jnp.float32)
        o_ref[...] = h[:, :3] + b3_ref[...]


def _fused_mlp(x, w1t, b1, w2t, b2, w3t, b3, *, tm=128, tk=3456):
    M, K = x.shape
    N1 = w1t.shape[1]
    N2 = w2t.shape[1]
    N3 = w3t.shape[1]
    tm = min(tm, _round_up(M, 16))
    Mp = _round_up(M, tm)
    if Mp != M:
        x = jnp.pad(x, ((0, Mp - M), (0, 0)))
    while K % tk:
        tk //= 2
    grid = (Mp // tm, K // tk)
    out = pl.pallas_call(
        _fc_kernel,
        out_shape=jax.ShapeDtypeStruct((Mp, 3), jnp.float32),
        grid=grid,
        in_specs=[
            pl.BlockSpec((tm, tk), lambda i, k: (i, k)),
            pl.BlockSpec((tk, N1), lambda i, k: (k, 0)),
            pl.BlockSpec((1, N1), lambda i, k: (0, 0)),
            pl.BlockSpec((N1, N2), lambda i, k: (0, 0)),
            pl.BlockSpec((1, N2), lambda i, k: (0, 0)),
            pl.BlockSpec((N2, N3), lambda i, k: (0, 0)),
            pl.BlockSpec((1, 3), lambda i, k: (0, 0)),
        ],
        out_specs=pl.BlockSpec((tm, 3), lambda i, k: (i, 0)),
        scratch_shapes=[pltpu.VMEM((tm, N1), jnp.float32)],
        compiler_params=pltpu.CompilerParams(
            dimension_semantics=("parallel", "arbitrary")),
    )(x, w1t, b1, w2t, b2, w3t, b3[:, :3])
    return out[:M]


def kernel(x, conv1_w, conv1_b, conv2_w, conv2_b, conv3_w, conv3_b,
           fc1_w, fc1_b, fc2_w, fc2_b, fc3_w, fc3_b):
    x, pre = _conv1_fused(x, conv1_w, conv1_b, cout=24)
    x = _convN_fused(x, conv2_w, conv2_b, cout=32, ksize=5, stride=2,
                     cin=24, prearranged=pre)
    x = _convN_fused(x, conv3_w, conv3_b, cout=64, ksize=3, stride=1)
    x = x.reshape(x.shape[0], -1)
    return _fused_mlp(x, fc1_w, fc1_b, fc2_w, fc2_b, fc3_w, fc3_b)


# speedup vs baseline: 1.4833x; 1.4833x over previous
import functools

import jax
import jax.numpy as jnp
from jax.experimental import pallas as pl
from jax.experimental.pallas import tpu as pltpu


def _round_up(v, m):
    return ((v + m - 1) // m) * m


def _convN_kernel(v_ref, w_ref, b_ref, o_ref, *, kh, kw, stride, cin, cout,
                  ho_n, wo_n):
    V = v_ref[0]
    S = stride * cin
    win = (kw - 1) * cin + cin
    ndj = (kw * cin + S - 1) // S
    pieces = []
    left = win
    for dj in range(ndj):
        wdt = min(S, left)
        pieces.append(V[:, dj:dj + wo_n, :wdt])
        left -= wdt
    T = jnp.concatenate(pieces, axis=-1)
    w = w_ref[...]
    b = b_ref[...]
    for ho in range(ho_n):
        lhs = jnp.concatenate([T[stride * ho + i] for i in range(kh)],
                              axis=-1)
        acc = jnp.dot(lhs, w, preferred_element_type=jnp.float32)
        out = jnp.maximum(acc[:, :cout] + b, 0.0)
        o_ref[0, ho] = out.astype(o_ref.dtype)


def _convN_fused(x, wmat, b2, *, cout, ksize, stride):
    n, h, w, c = x.shape
    ho = (h - ksize) // stride + 1
    wo = (w - ksize) // stride + 1
    if w % stride:
        x = jnp.pad(x, ((0, 0), (0, 0), (0, stride - w % stride), (0, 0)))
        w += stride - w % stride
    w2 = w // stride
    v = x.reshape(n, h, w2, stride * c)
    kern = functools.partial(_convN_kernel, kh=ksize, kw=ksize, stride=stride,
                             cin=c, cout=cout, ho_n=ho, wo_n=wo)
    return pl.pallas_call(
        kern,
        out_shape=jax.ShapeDtypeStruct((n, ho, wo, cout), jnp.bfloat16),
        grid=(n,),
        in_specs=[
            pl.BlockSpec((1, h, w2, stride * c), lambda i: (i, 0, 0, 0)),
            pl.BlockSpec(wmat.shape, lambda i: (0, 0)),
            pl.BlockSpec((1, cout), lambda i: (0, 0)),
        ],
        out_specs=pl.BlockSpec((1, ho, wo, cout), lambda i: (i, 0, 0, 0)),
        compiler_params=pltpu.CompilerParams(
            dimension_semantics=("parallel",)),
    )(v, wmat, b2[:, :cout])


def _conv1_kernel(xm_ref, g_ref, w_ref, b_ref, o_ref, *, ho_n, wo_n, cin,
                  cout):
    xm = xm_ref[0].astype(jnp.bfloat16)
    xt = jax.lax.dot_general(xm, g_ref[...], (((0,), (0,)), ((), ())),
                             preferred_element_type=jnp.float32)
    xp = xt.reshape(xt.shape[0] // 2, 2, xt.shape[1])
    w = w_ref[...]
    b = b_ref[...]
    kwin = 5 * cin
    for ho in range(ho_n):
        base = 2 * cin * ho
        pieces = []
        for j in range(5):
            p, dj = j % 2, j // 2
            pieces.append(xp[dj:dj + wo_n, p, base:base + kwin])
        lhs = jnp.concatenate(pieces, axis=-1)
        acc = jnp.dot(lhs.astype(jnp.bfloat16), w,
                      preferred_element_type=jnp.float32)
        out = jnp.maximum(acc[:, :cout] + b, 0.0)
        o_ref[0, ho] = out.astype(o_ref.dtype)


def _conv1_fused(x_nchw, wmat, b2, *, cout=24):
    n, c, h, w = x_nchw.shape
    ho = (h - 5) // 2 + 1
    wo = (w - 5) // 2 + 1
    xm = x_nchw.reshape(n, c * h, w)
    if w % 2:
        xm = jnp.pad(xm, ((0, 0), (0, 0), (0, 1)))
        w += 1
    cc = jnp.arange(c)
    hh = jnp.arange(h)
    g = ((hh[None, :, None, None] == hh[None, None, :, None])
         & (cc[:, None, None, None] == cc[None, None, None, :]))
    g = g.reshape(c * h, h * c).astype(jnp.bfloat16)
    wmat = (wmat.reshape(5, 5, c, wmat.shape[1])
            .transpose(1, 0, 2, 3).reshape(25 * c, wmat.shape[1]))
    kern = functools.partial(_conv1_kernel, ho_n=ho, wo_n=wo, cin=c,
                             cout=cout)
    return pl.pallas_call(
        kern,
        out_shape=jax.ShapeDtypeStruct((n, ho, wo, cout), jnp.bfloat16),
        grid=(n,),
        in_specs=[
            pl.BlockSpec((1, c * h, w), lambda i: (i, 0, 0)),
            pl.BlockSpec(g.shape, lambda i: (0, 0)),
            pl.BlockSpec(wmat.shape, lambda i: (0, 0)),
            pl.BlockSpec((1, cout), lambda i: (0, 0)),
        ],
        out_specs=pl.BlockSpec((1, ho, wo, cout), lambda i: (i, 0, 0, 0)),
        compiler_params=pltpu.CompilerParams(
            dimension_semantics=("parallel",)),
    )(xm, g, wmat, b2[:, :cout])


def _fc_kernel(x_ref, w1_ref, b1_ref, w2_ref, b2_ref, w3_ref, b3_ref,
               o_ref, acc_ref):
    @pl.when(pl.program_id(1) == 0)
    def _():
        acc_ref[...] = jnp.zeros_like(acc_ref)

    acc_ref[...] += jnp.dot(x_ref[...], w1_ref[...],
                            preferred_element_type=jnp.float32)

    @pl.when(pl.program_id(1) == pl.num_programs(1) - 1)
    def _():
        h = jnp.maximum(acc_ref[...] + b1_ref[...], 0.0).astype(jnp.bfloat16)
        h = jnp.dot(h, w2_ref[...], preferred_element_type=jnp.float32)
        h = jnp.maximum(h + b2_ref[...], 0.0).astype(jnp.bfloat16)
        h = jnp.dot(h, w3_ref[...], preferred_element_type=jnp.float32)
        o_ref[...] = h[:, :3] + b3_ref[...]


def _fused_mlp(x, w1t, b1, w2t, b2, w3t, b3, *, tm=128, tk=3456):
    M, K = x.shape
    N1 = w1t.shape[1]
    N2 = w2t.shape[1]
    N3 = w3t.shape[1]
    tm = min(tm, _round_up(M, 16))
    Mp = _round_up(M, tm)
    if Mp != M:
        x = jnp.pad(x, ((0, Mp - M), (0, 0)))
    while K % tk:
        tk //= 2
    grid = (Mp // tm, K // tk)
    out = pl.pallas_call(
        _fc_kernel,
        out_shape=jax.ShapeDtypeStruct((Mp, 3), jnp.float32),
        grid=grid,
        in_specs=[
            pl.BlockSpec((tm, tk), lambda i, k: (i, k)),
            pl.BlockSpec((tk, N1), lambda i, k: (k, 0)),
            pl.BlockSpec((1, N1), lambda i, k: (0, 0)),
            pl.BlockSpec((N1, N2), lambda i, k: (0, 0)),
            pl.BlockSpec((1, N2), lambda i, k: (0, 0)),
            pl.BlockSpec((N2, N3), lambda i, k: (0, 0)),
            pl.BlockSpec((1, 3), lambda i, k: (0, 0)),
        ],
        out_specs=pl.BlockSpec((tm, 3), lambda i, k: (i, 0)),
        scratch_shapes=[pltpu.VMEM((tm, N1), jnp.float32)],
        compiler_params=pltpu.CompilerParams(
            dimension_semantics=("parallel", "arbitrary")),
    )(x, w1t, b1, w2t, b2, w3t, b3[:, :3])
    return out[:M]


def kernel(x, conv1_w, conv1_b, conv2_w, conv2_b, conv3_w, conv3_b,
           fc1_w, fc1_b, fc2_w, fc2_b, fc3_w, fc3_b):
    x = _conv1_fused(x, conv1_w, conv1_b, cout=24)
    x = _convN_fused(x, conv2_w, conv2_b, cout=32, ksize=5, stride=2)
    x = _convN_fused(x, conv3_w, conv3_b, cout=64, ksize=3, stride=1)
    x = x.reshape(x.shape[0], -1)
    return _fused_mlp(x, fc1_w, fc1_b, fc2_w, fc2_b, fc3_w, fc3_b)
